# baseline (device time: 177108 ns/iter reference)
import jax
import jax.numpy as jnp
from jax import lax
from jax.experimental import pallas as pl
from jax.experimental.pallas import tpu as pltpu

N_DEV = 16
B, SQ, SKV, HQ_TOTAL, DH = 2, 512, 512, 128, 64
H_PER = HQ_TOTAL // N_DEV
D_MODEL = 768
ROWS = B * SQ

_HALVES = [512, 256, 128, 64]
_RBUF_OFFS = [0, 512, 768, 896]


def _fused(x, wqT, k_ext, v_ext, wo):

    def body(x_ref, wqT_ref, ke_ref, ve_ref, wo_ref, out_ref,
             kscr, vscr, acc_ref, rbuf_ref,
             ksem, vsem, rs_send, rs_recv, ag_send, ag_recv):
        my = lax.axis_index("i")
        j = lax.rem(my, 4)
        z = lax.div(my, 4)
        bits = [
            jnp.where((j == 1) | (j == 2), 1, 0),
            jnp.where(j >= 2, 1, 0),
            lax.rem(z, 2),
            lax.div(z, 2),
        ]
        partners = [my ^ 1, my ^ 3, my ^ 4, my ^ 8]

        barrier_sem = pltpu.get_barrier_semaphore()
        for p in partners:
            pl.semaphore_signal(
                barrier_sem, inc=1,
                device_id=(p,), device_id_type=pl.DeviceIdType.MESH,
            )

        kcopies, vcopies = [], []
        for b in range(B):
            for h in range(H_PER):
                bh = b * H_PER + h
                hh = my * H_PER + h
                kc = pltpu.make_async_copy(
                    ke_ref.at[b, :, hh, :], kscr.at[bh], ksem.at[bh]
                )
                vc = pltpu.make_async_copy(
                    ve_ref.at[b, :, hh, :], vscr.at[bh], vsem.at[bh]
                )
                kc.start()
                vc.start()
                kcopies.append(kc)
                vcopies.append(vc)

        qi = lax.broadcasted_iota(jnp.int32, (SQ, SKV), 0)
        ki = lax.broadcasted_iota(jnp.int32, (SQ, SKV), 1)
        mask = (jnp.abs(qi - ki) <= 128) | (ki < 32) | (qi < 32)

        for b in range(B):
            for h in range(H_PER):
                bh = b * H_PER + h
                kcopies[bh].wait()
                vcopies[bh].wait()
                qh = lax.dot_general(
                    x_ref[b], wqT_ref[h], (((1,), (1,)), ((), ())),
                    preferred_element_type=jnp.float32,
                ).astype(jnp.bfloat16)
                kh = kscr[bh].astype(jnp.bfloat16)
                vh = vscr[bh].astype(jnp.bfloat16)
                s = lax.dot_general(
                    qh, kh, (((1,), (1,)), ((), ())),
                    preferred_element_type=jnp.float32,
                ) * 0.125
                s = jnp.where(mask, s, -1e9)
                m = jnp.max(s, axis=1, keepdims=True)
                e = jnp.exp(s - m)
                w = (e / jnp.sum(e, axis=1, keepdims=True)).astype(
                    jnp.bfloat16
                )
                ctx = lax.dot_general(
                    w, vh, (((1,), (0,)), ((), ())),
                    preferred_element_type=jnp.float32,
                ).astype(jnp.bfloat16)
                pw = lax.dot_general(
                    ctx, wo_ref[h * DH:(h + 1) * DH, :],
                    (((1,), (0,)), ((), ())),
                    preferred_element_type=jnp.float32,
                )
                if h == 0:
                    acc_ref[b * SQ:(b + 1) * SQ, :] = pw
                else:
                    acc_ref[b * SQ:(b + 1) * SQ, :] = (
                        acc_ref[b * SQ:(b + 1) * SQ, :] + pw
                    )

        out_ref[:, :] = acc_ref[:, :].astype(jnp.bfloat16)

        pl.semaphore_wait(barrier_sem, len(partners))

        S = jnp.int32(0)
        for k in range(4):
            h = _HALVES[k]
            send_off = S + (1 - bits[k]) * h
            keep_off = S + bits[k] * h
            rdma = pltpu.make_async_remote_copy(
                src_ref=out_ref.at[pl.ds(send_off, h)],
                dst_ref=rbuf_ref.at[pl.ds(_RBUF_OFFS[k], h)],
                send_sem=rs_send.at[k],
                recv_sem=rs_recv.at[k],
                device_id=(partners[k],),
                device_id_type=pl.DeviceIdType.MESH,
            )
            rdma.start()
            rdma.wait()
            out_ref[pl.ds(keep_off, h), :] = (
                out_ref[pl.ds(keep_off, h), :]
                + rbuf_ref[pl.ds(_RBUF_OFFS[k], h), :]
            )
            S = keep_off

        for k in (3, 2, 1, 0):
            g = _HALVES[k]
            rdma = pltpu.make_async_remote_copy(
                src_ref=out_ref.at[pl.ds(S, g)],
                dst_ref=out_ref.at[pl.ds(S, g)],
                send_sem=ag_send.at[k],
                recv_sem=ag_recv.at[k],
                device_id=(partners[k],),
                device_id_type=pl.DeviceIdType.MESH,
            )
            rdma.start()
            rdma.wait()
            S = S - bits[k] * g

    return pl.pallas_call(
        body,
        out_shape=jax.ShapeDtypeStruct((ROWS, D_MODEL), jnp.bfloat16),
        in_specs=[
            pl.BlockSpec(memory_space=pltpu.VMEM),
            pl.BlockSpec(memory_space=pltpu.VMEM),
            pl.BlockSpec(memory_space=pltpu.MemorySpace.HBM),
            pl.BlockSpec(memory_space=pltpu.MemorySpace.HBM),
            pl.BlockSpec(memory_space=pltpu.VMEM),
        ],
        out_specs=pl.BlockSpec(memory_space=pltpu.VMEM),
        scratch_shapes=[
            pltpu.VMEM((B * H_PER, SKV, DH), jnp.float32),
            pltpu.VMEM((B * H_PER, SKV, DH), jnp.float32),
            pltpu.VMEM((ROWS, D_MODEL), jnp.float32),
            pltpu.VMEM((ROWS, D_MODEL), jnp.bfloat16),
            pltpu.SemaphoreType.DMA((B * H_PER,)),
            pltpu.SemaphoreType.DMA((B * H_PER,)),
            pltpu.SemaphoreType.DMA((4,)),
            pltpu.SemaphoreType.DMA((4,)),
            pltpu.SemaphoreType.DMA((4,)),
            pltpu.SemaphoreType.DMA((4,)),
        ],
        compiler_params=pltpu.CompilerParams(collective_id=0),
    )(x, wqT, k_ext, v_ext, wo)


def kernel(x, Wq, K_ext, V_ext, Wo):
    xb = x.astype(jnp.bfloat16)
    wqT = jnp.transpose(Wq, (1, 0)).reshape(H_PER, DH, D_MODEL).astype(
        jnp.bfloat16
    )
    out = _fused(xb, wqT, K_ext, V_ext, Wo.astype(jnp.bfloat16))
    return out.reshape(B, SQ, D_MODEL)
